# baseline (speedup 1.0000x reference)
import functools

import jax
import jax.numpy as jnp
from jax import lax
from jax.experimental import pallas as pl
from jax.experimental.pallas import tpu as pltpu
from jax.experimental.pallas import tpu_sc as plsc

N = 10000
E = 320000
D = 128
ALPHA = 0.1

NC = 2
NS = 16
NW = NC * NS
C = 100
CHUNKS_PER_TILE = E // (NW * C)
ROWS_PER_TILE = N // NS
WB = 632
HOP = (C // 8) * 8
RB = 5
RBS = N // RB

_mesh = functools.partial(
    plsc.VectorSubcoreMesh, core_axis_name="c", subcore_axis_name="s")


def _dinv_of(p_ref):
    deg = 1.0 + p_ref[:, 0] + p_ref[:, 1]
    return lax.rsqrt(deg)[:, None]


def _enc_body(x_ref, tw_ref, tb_ref, qe_ref, qw_ref, qb_ref, p_ref,
              h_ref, q_ref, g_ref):
    h = lax.dot_general(
        x_ref[...], tw_ref[...], (((1,), (1,)), ((), ())),
        preferred_element_type=jnp.float32) + tb_ref[...]
    h_ref[...] = h
    g_ref[...] = h * _dinv_of(p_ref)
    q_ref[...] = lax.dot_general(
        qe_ref[...], qw_ref[...], (((1,), (1,)), ((), ())),
        preferred_element_type=jnp.float32) + qb_ref[...]


def _encoder_prescale(x, t_W, t_b, q_emb, q_W, q_b, degp):
    h, q, g = pl.pallas_call(
        _enc_body,
        grid=(RB,),
        in_specs=[
            pl.BlockSpec((RBS, D), lambda i: (i, 0)),
            pl.BlockSpec((D, D), lambda i: (0, 0)),
            pl.BlockSpec((1, D), lambda i: (0, 0)),
            pl.BlockSpec((1, D), lambda i: (0, 0)),
            pl.BlockSpec((D, D), lambda i: (0, 0)),
            pl.BlockSpec((1, D), lambda i: (0, 0)),
            pl.BlockSpec((RBS, 2), lambda i: (i, 0)),
        ],
        out_specs=[
            pl.BlockSpec((RBS, D), lambda i: (i, 0)),
            pl.BlockSpec((1, D), lambda i: (0, 0)),
            pl.BlockSpec((RBS, D), lambda i: (i, 0)),
        ],
        out_shape=[
            jax.ShapeDtypeStruct((N, D), jnp.float32),
            jax.ShapeDtypeStruct((1, D), jnp.float32),
            jax.ShapeDtypeStruct((N, D), jnp.float32),
        ],
    )(x, t_W, t_b.reshape(1, D), q_emb.reshape(1, D), q_W,
      q_b.reshape(1, D), degp)
    return h, q.reshape(D), g


def _blend_body(final, p_ref, a_ref, g_ref, h0_ref, o_ref):
    dinv = _dinv_of(p_ref)
    ah = dinv * (a_ref[0] + a_ref[1] + g_ref[...])
    h = (1.0 - ALPHA) * ah + ALPHA * h0_ref[...]
    o_ref[...] = h if final else h * dinv


def _blend(degp, a, g, h0, final):
    return pl.pallas_call(
        functools.partial(_blend_body, final),
        grid=(RB,),
        in_specs=[
            pl.BlockSpec((RBS, 2), lambda i: (i, 0)),
            pl.BlockSpec((2, RBS, D), lambda i: (0, i, 0)),
            pl.BlockSpec((RBS, D), lambda i: (i, 0)),
            pl.BlockSpec((RBS, D), lambda i: (i, 0)),
        ],
        out_specs=pl.BlockSpec((RBS, D), lambda i: (i, 0)),
        out_shape=jax.ShapeDtypeStruct((N, D), jnp.float32),
    )(degp, a, g, h0)


def _deg_body(col_hbm, out_hbm, colbuf, ones_v, zbuf, acc, sem):
    cid = lax.axis_index("c")
    sid = lax.axis_index("s")

    for k in range(8):
        ones_v[pl.ds(k * 16, 16)] = jnp.ones((16,), jnp.float32)

    def zb_store(k, carry):
        zbuf[pl.ds(k * 16, 16)] = jnp.zeros((16,), jnp.float32)
        return carry
    lax.fori_loop(0, 40, zb_store, 0)

    @pl.when(sid == 0)
    def _():
        def zcpy(k, carry):
            pltpu.sync_copy(zbuf, acc.at[pl.ds(k * 640, 640)])
            return carry
        lax.fori_loop(0, N // 640, zcpy, 0)
        pltpu.sync_copy(zbuf.at[pl.ds(0, N % 640)],
                        acc.at[pl.ds((N // 640) * 640, N % 640)])

    plsc.subcore_barrier()

    wid = cid * NS + sid
    pltpu.sync_copy(col_hbm.at[wid], colbuf)

    def body(j, carry):
        pltpu.async_copy(ones_v.at[pl.ds(0, C)], acc.at[colbuf.at[j]],
                         sem, add=True)
        return carry
    lax.fori_loop(0, CHUNKS_PER_TILE, body, 0)

    def drain(j, carry):
        pltpu.make_async_copy(ones_v.at[pl.ds(0, C)],
                              acc.at[colbuf.at[j]], sem).wait()
        return carry
    lax.fori_loop(0, CHUNKS_PER_TILE, drain, 0)

    plsc.subcore_barrier()

    start = sid * ROWS_PER_TILE
    s8 = (start // 8) * 8
    pltpu.sync_copy(acc.at[pl.ds(s8, 632)], zbuf.at[pl.ds(0, 632)])
    pltpu.sync_copy(zbuf.at[pl.ds(0, 632)],
                    out_hbm.at[pl.ds(cid * N + s8, 632)])


def _deg_sc(col2d):
    k = pl.kernel(
        _deg_body,
        out_type=jax.ShapeDtypeStruct((NC * N,), jnp.float32),
        mesh=_mesh(),
        scratch_types=[
            pltpu.VMEM((CHUNKS_PER_TILE, C), jnp.int32),
            pltpu.VMEM((128,), jnp.float32),
            pltpu.VMEM((640,), jnp.float32),
            pltpu.VMEM_SHARED((N,), jnp.float32),
            pltpu.SemaphoreType.DMA,
        ],
    )
    return k(col2d)


def _prop_body(g_hbm, e4_hbm, out_hbm,
               idx_0, idx_1, idx_2, idx_3, idx_4, idx_5,
               rows_0, rows_1, rows_2, acc,
               sg0, sg1, sg2, si0, si1, si2, si3, si4, si5,
               ss0, ss1, ss2):
    cid = lax.axis_index("c")
    sid = lax.axis_index("s")
    wid = cid * NS + sid
    ibufs = ((idx_0, si0), (idx_1, si1), (idx_2, si2),
             (idx_3, si3), (idx_4, si4), (idx_5, si5))
    rbufs = ((rows_0, sg0, ss0), (rows_1, sg1, ss1), (rows_2, sg2, ss2))
    nch = CHUNKS_PER_TILE
    r8 = (sid * ROWS_PER_TILE // 8) * 8

    def zrow(i, carry):
        for k in range(D // 16):
            rows_2[i, pl.ds(k * 16, 16)] = jnp.zeros((16,), jnp.float32)
        return carry
    lax.fori_loop(0, C, zrow, 0)

    pltpu.sync_copy(e4_hbm.at[wid, 0], idx_0)
    pltpu.async_copy(g_hbm.at[idx_0.at[0]], rows_0, sg0)
    pltpu.async_copy(e4_hbm.at[wid, 1], idx_1, si1)
    pltpu.make_async_copy(e4_hbm.at[wid, 1], idx_1, si1).wait()
    pltpu.async_copy(g_hbm.at[idx_1.at[0]], rows_1, sg1)
    pltpu.async_copy(e4_hbm.at[wid, 2], idx_2, si2)
    pltpu.async_copy(e4_hbm.at[wid, 3], idx_3, si3)

    for k in range(WB // HOP):
        pltpu.async_copy(rows_2.at[pl.ds(0, HOP), :],
                         acc.at[pl.ds(r8 + k * HOP, HOP), :], ss0)
    pltpu.async_copy(rows_2.at[pl.ds(0, WB % HOP), :],
                     acc.at[pl.ds(r8 + (WB // HOP) * HOP, WB % HOP), :],
                     ss0)
    for k in range(WB // HOP):
        pltpu.make_async_copy(rows_2.at[pl.ds(0, HOP), :],
                              acc.at[pl.ds(r8 + k * HOP, HOP), :],
                              ss0).wait()
    pltpu.make_async_copy(
        rows_2.at[pl.ds(0, WB % HOP), :],
        acc.at[pl.ds(r8 + (WB // HOP) * HOP, WB % HOP), :], ss0).wait()

    plsc.subcore_barrier()

    def chunk_step(j, k6):
        idx_c, _ = ibufs[k6]
        rows_c, sg_c, ss_c = rbufs[k6 % 3]
        idx_n, si_n = ibufs[(k6 + 2) % 6]
        rows_n, sg_n, ss_n = rbufs[(k6 + 2) % 3]
        idx_p, si_p = ibufs[(k6 + 4) % 6]
        idx_o, _ = ibufs[(k6 + 1) % 6]
        pltpu.make_async_copy(g_hbm.at[idx_c.at[0]], rows_c, sg_c).wait()

        @pl.when(j < nch - 2)
        def _():
            pltpu.make_async_copy(
                e4_hbm.at[wid, j + 2], idx_n, si_n).wait()

            @pl.when(j >= 1)
            def _():
                pltpu.make_async_copy(
                    rows_n, acc.at[ibufs[(k6 + 5) % 6][0].at[1]],
                    ss_n).wait()
            pltpu.async_copy(g_hbm.at[idx_n.at[0]], rows_n, sg_n)
        pltpu.async_copy(rows_c, acc.at[idx_c.at[1]], ss_c, add=True)

        @pl.when(j < nch - 4)
        def _():
            pltpu.async_copy(e4_hbm.at[wid, j + 4], idx_p, si_p)

    def body(t, carry):
        for k in range(6):
            chunk_step(6 * t + k, k)
        return carry
    lax.fori_loop(0, nch // 6, body, 0)

    for j in range(nch - nch % 6, nch):
        chunk_step(j, j % 6)

    for j in (nch - 3, nch - 2, nch - 1):
        idx_d, _ = ibufs[j % 6]
        rows_d, _, ss_d = rbufs[j % 3]
        pltpu.make_async_copy(rows_d, acc.at[idx_d.at[1]], ss_d).wait()

    plsc.subcore_barrier()

    nhop = WB // HOP
    wbufs = (rows_0, rows_1)
    wsems = (sg0, sg1)

    def wbhop(k, nrows):
        b = k % 2
        if k >= 2:
            pltpu.make_async_copy(
                wbufs[b].at[pl.ds(0, HOP), :],
                out_hbm.at[cid, pl.ds(r8 + (k - 2) * HOP, HOP), :],
                wsems[b]).wait()
        pltpu.sync_copy(acc.at[pl.ds(r8 + k * HOP, nrows), :],
                        wbufs[b].at[pl.ds(0, nrows), :])
        pltpu.async_copy(wbufs[b].at[pl.ds(0, nrows), :],
                         out_hbm.at[cid, pl.ds(r8 + k * HOP, nrows), :],
                         wsems[b])

    for k in range(nhop):
        wbhop(k, HOP)
    wbhop(nhop, WB % HOP)
    for k in (nhop - 1, nhop):
        b = k % 2
        nrows = HOP if k < nhop else WB % HOP
        pltpu.make_async_copy(
            wbufs[b].at[pl.ds(0, nrows), :],
            out_hbm.at[cid, pl.ds(r8 + k * HOP, nrows), :],
            wsems[b]).wait()


def _prop_sc(g, e4):
    k = pl.kernel(
        _prop_body,
        out_type=jax.ShapeDtypeStruct((NC, N, D), jnp.float32),
        mesh=_mesh(),
        scratch_types=(
            [pltpu.VMEM((2, C), jnp.int32)] * 6
            + [pltpu.VMEM((C, D), jnp.float32)] * 3
            + [pltpu.VMEM_SHARED((N, D), jnp.float32)]
            + [pltpu.SemaphoreType.DMA] * 12
        ),
    )
    return k(g, e4)


def kernel(x, edge_index, q_emb, t_W, t_b, q_W, q_b):
    row3 = edge_index[0].reshape(NW, CHUNKS_PER_TILE, C)
    col3 = edge_index[1].reshape(NW, CHUNKS_PER_TILE, C)
    e4 = jnp.stack([row3, col3], axis=2)

    degp = _deg_sc(col3).reshape(NC, N).T
    h0, ques, g1 = _encoder_prescale(x, t_W, t_b, q_emb, q_W, q_b, degp)
    a1 = _prop_sc(g1, e4)
    g2 = _blend(degp, a1, g1, h0, final=False)
    a2 = _prop_sc(g2, e4)
    h2 = _blend(degp, a2, g2, h0, final=True)
    return ques, h2


# speedup vs baseline: 1.0103x; 1.0103x over previous
import functools

import jax
import jax.numpy as jnp
from jax import lax
from jax.experimental import pallas as pl
from jax.experimental.pallas import tpu as pltpu
from jax.experimental.pallas import tpu_sc as plsc

N = 10000
E = 320000
D = 128
ALPHA = 0.1

NC = 2
NS = 16
NW = NC * NS
C = 100
CHUNKS_PER_TILE = E // (NW * C)
ROWS_PER_TILE = N // NS
WB = 632
HOP = (C // 8) * 8
RB = 5
RBS = N // RB

_mesh = functools.partial(
    plsc.VectorSubcoreMesh, core_axis_name="c", subcore_axis_name="s")


def _dinv_of(p_ref):
    deg = 1.0 + p_ref[:, 0] + p_ref[:, 1]
    return lax.rsqrt(deg)[:, None]


def _enc_body(x_ref, tw_ref, tb_ref, qe_ref, qw_ref, qb_ref, p_ref,
              h_ref, q_ref, g_ref):
    h = lax.dot_general(
        x_ref[...], tw_ref[...], (((1,), (1,)), ((), ())),
        preferred_element_type=jnp.float32) + tb_ref[...]
    h_ref[...] = h
    g_ref[...] = h * _dinv_of(p_ref)
    q_ref[...] = lax.dot_general(
        qe_ref[...], qw_ref[...], (((1,), (1,)), ((), ())),
        preferred_element_type=jnp.float32) + qb_ref[...]


def _encoder_prescale(x, t_W, t_b, q_emb, q_W, q_b, degp):
    h, q, g = pl.pallas_call(
        _enc_body,
        grid=(RB,),
        in_specs=[
            pl.BlockSpec((RBS, D), lambda i: (i, 0)),
            pl.BlockSpec((D, D), lambda i: (0, 0)),
            pl.BlockSpec((1, D), lambda i: (0, 0)),
            pl.BlockSpec((1, D), lambda i: (0, 0)),
            pl.BlockSpec((D, D), lambda i: (0, 0)),
            pl.BlockSpec((1, D), lambda i: (0, 0)),
            pl.BlockSpec((RBS, 2), lambda i: (i, 0)),
        ],
        out_specs=[
            pl.BlockSpec((RBS, D), lambda i: (i, 0)),
            pl.BlockSpec((1, D), lambda i: (0, 0)),
            pl.BlockSpec((RBS, D), lambda i: (i, 0)),
        ],
        out_shape=[
            jax.ShapeDtypeStruct((N, D), jnp.float32),
            jax.ShapeDtypeStruct((1, D), jnp.float32),
            jax.ShapeDtypeStruct((N, D), jnp.float32),
        ],
    )(x, t_W, t_b.reshape(1, D), q_emb.reshape(1, D), q_W,
      q_b.reshape(1, D), degp)
    return h, q.reshape(D), g


def _blend_body(final, p_ref, a_ref, g_ref, h0_ref, o_ref):
    dinv = _dinv_of(p_ref)
    ah = dinv * (a_ref[0] + a_ref[1] + g_ref[...])
    h = (1.0 - ALPHA) * ah + ALPHA * h0_ref[...]
    o_ref[...] = h if final else h * dinv


def _blend(degp, a, g, h0, final):
    return pl.pallas_call(
        functools.partial(_blend_body, final),
        grid=(RB,),
        in_specs=[
            pl.BlockSpec((RBS, 2), lambda i: (i, 0)),
            pl.BlockSpec((2, RBS, D), lambda i: (0, i, 0)),
            pl.BlockSpec((RBS, D), lambda i: (i, 0)),
            pl.BlockSpec((RBS, D), lambda i: (i, 0)),
        ],
        out_specs=pl.BlockSpec((RBS, D), lambda i: (i, 0)),
        out_shape=jax.ShapeDtypeStruct((N, D), jnp.float32),
    )(degp, a, g, h0)


def _deg_body(col_hbm, out_hbm, colbuf, ones_v, zbuf, acc, sem):
    cid = lax.axis_index("c")
    sid = lax.axis_index("s")
    wid = cid * NS + sid

    pltpu.async_copy(col_hbm.at[wid], colbuf, sem)

    for k in range(8):
        ones_v[pl.ds(k * 16, 16)] = jnp.ones((16,), jnp.float32)

    def zb_store(k, carry):
        zbuf[pl.ds(k * 16, 16)] = jnp.zeros((16,), jnp.float32)
        return carry
    lax.fori_loop(0, 40, zb_store, 0)

    s8w = (sid * ROWS_PER_TILE // 8) * 8
    pltpu.sync_copy(zbuf.at[pl.ds(0, WB)], acc.at[pl.ds(s8w, WB)])

    plsc.subcore_barrier()

    pltpu.make_async_copy(col_hbm.at[wid], colbuf, sem).wait()

    def body(j, carry):
        pltpu.async_copy(ones_v.at[pl.ds(0, C)], acc.at[colbuf.at[j]],
                         sem, add=True)
        return carry
    lax.fori_loop(0, CHUNKS_PER_TILE, body, 0)

    def drain(j, carry):
        pltpu.make_async_copy(ones_v.at[pl.ds(0, C)],
                              acc.at[colbuf.at[j]], sem).wait()
        return carry
    lax.fori_loop(0, CHUNKS_PER_TILE, drain, 0)

    plsc.subcore_barrier()

    start = sid * ROWS_PER_TILE
    s8 = (start // 8) * 8
    pltpu.sync_copy(acc.at[pl.ds(s8, 632)], zbuf.at[pl.ds(0, 632)])
    pltpu.sync_copy(zbuf.at[pl.ds(0, 632)],
                    out_hbm.at[pl.ds(cid * N + s8, 632)])


def _deg_sc(col2d):
    k = pl.kernel(
        _deg_body,
        out_type=jax.ShapeDtypeStruct((NC * N,), jnp.float32),
        mesh=_mesh(),
        scratch_types=[
            pltpu.VMEM((CHUNKS_PER_TILE, C), jnp.int32),
            pltpu.VMEM((128,), jnp.float32),
            pltpu.VMEM((640,), jnp.float32),
            pltpu.VMEM_SHARED((N,), jnp.float32),
            pltpu.SemaphoreType.DMA,
        ],
    )
    return k(col2d)


def _prop_body(g_hbm, e4_hbm, out_hbm,
               idx_0, idx_1, idx_2, idx_3, idx_4, idx_5,
               rows_0, rows_1, rows_2, acc,
               sg0, sg1, sg2, si0, si1, si2, si3, si4, si5,
               ss0, ss1, ss2):
    cid = lax.axis_index("c")
    sid = lax.axis_index("s")
    wid = cid * NS + sid
    ibufs = ((idx_0, si0), (idx_1, si1), (idx_2, si2),
             (idx_3, si3), (idx_4, si4), (idx_5, si5))
    rbufs = ((rows_0, sg0, ss0), (rows_1, sg1, ss1), (rows_2, sg2, ss2))
    nch = CHUNKS_PER_TILE
    r8 = (sid * ROWS_PER_TILE // 8) * 8

    pltpu.async_copy(e4_hbm.at[wid, 0], idx_0, si0)
    pltpu.async_copy(e4_hbm.at[wid, 1], idx_1, si1)
    pltpu.async_copy(e4_hbm.at[wid, 2], idx_2, si2)
    pltpu.async_copy(e4_hbm.at[wid, 3], idx_3, si3)

    def zrow(i, carry):
        for k in range(D // 16):
            rows_2[i, pl.ds(k * 16, 16)] = jnp.zeros((16,), jnp.float32)
        return carry
    lax.fori_loop(0, C, zrow, 0)

    pltpu.make_async_copy(e4_hbm.at[wid, 0], idx_0, si0).wait()
    pltpu.async_copy(g_hbm.at[idx_0.at[0]], rows_0, sg0)
    pltpu.make_async_copy(e4_hbm.at[wid, 1], idx_1, si1).wait()
    pltpu.async_copy(g_hbm.at[idx_1.at[0]], rows_1, sg1)

    for k in range(WB // HOP):
        pltpu.async_copy(rows_2.at[pl.ds(0, HOP), :],
                         acc.at[pl.ds(r8 + k * HOP, HOP), :], ss0)
    pltpu.async_copy(rows_2.at[pl.ds(0, WB % HOP), :],
                     acc.at[pl.ds(r8 + (WB // HOP) * HOP, WB % HOP), :],
                     ss0)
    for k in range(WB // HOP):
        pltpu.make_async_copy(rows_2.at[pl.ds(0, HOP), :],
                              acc.at[pl.ds(r8 + k * HOP, HOP), :],
                              ss0).wait()
    pltpu.make_async_copy(
        rows_2.at[pl.ds(0, WB % HOP), :],
        acc.at[pl.ds(r8 + (WB // HOP) * HOP, WB % HOP), :], ss0).wait()

    plsc.subcore_barrier()

    def chunk_step(j, k6):
        idx_c, _ = ibufs[k6]
        rows_c, sg_c, ss_c = rbufs[k6 % 3]
        idx_n, si_n = ibufs[(k6 + 2) % 6]
        rows_n, sg_n, ss_n = rbufs[(k6 + 2) % 3]
        idx_p, si_p = ibufs[(k6 + 4) % 6]
        idx_o, _ = ibufs[(k6 + 1) % 6]
        pltpu.make_async_copy(g_hbm.at[idx_c.at[0]], rows_c, sg_c).wait()

        @pl.when(j < nch - 2)
        def _():
            pltpu.make_async_copy(
                e4_hbm.at[wid, j + 2], idx_n, si_n).wait()

            @pl.when(j >= 1)
            def _():
                pltpu.make_async_copy(
                    rows_n, acc.at[ibufs[(k6 + 5) % 6][0].at[1]],
                    ss_n).wait()
            pltpu.async_copy(g_hbm.at[idx_n.at[0]], rows_n, sg_n)
        pltpu.async_copy(rows_c, acc.at[idx_c.at[1]], ss_c, add=True)

        @pl.when(j < nch - 4)
        def _():
            pltpu.async_copy(e4_hbm.at[wid, j + 4], idx_p, si_p)

    def body(t, carry):
        for k in range(6):
            chunk_step(6 * t + k, k)
        return carry
    lax.fori_loop(0, nch // 6, body, 0)

    for j in range(nch - nch % 6, nch):
        chunk_step(j, j % 6)

    for j in (nch - 3, nch - 2, nch - 1):
        idx_d, _ = ibufs[j % 6]
        rows_d, _, ss_d = rbufs[j % 3]
        pltpu.make_async_copy(rows_d, acc.at[idx_d.at[1]], ss_d).wait()

    plsc.subcore_barrier()

    nhop = WB // HOP
    wbufs = (rows_0, rows_1)
    wsems = (sg0, sg1)

    def wbhop(k, nrows):
        b = k % 2
        if k >= 2:
            pltpu.make_async_copy(
                wbufs[b].at[pl.ds(0, HOP), :],
                out_hbm.at[cid, pl.ds(r8 + (k - 2) * HOP, HOP), :],
                wsems[b]).wait()
        pltpu.sync_copy(acc.at[pl.ds(r8 + k * HOP, nrows), :],
                        wbufs[b].at[pl.ds(0, nrows), :])
        pltpu.async_copy(wbufs[b].at[pl.ds(0, nrows), :],
                         out_hbm.at[cid, pl.ds(r8 + k * HOP, nrows), :],
                         wsems[b])

    for k in range(nhop):
        wbhop(k, HOP)
    wbhop(nhop, WB % HOP)
    for k in (nhop - 1, nhop):
        b = k % 2
        nrows = HOP if k < nhop else WB % HOP
        pltpu.make_async_copy(
            wbufs[b].at[pl.ds(0, nrows), :],
            out_hbm.at[cid, pl.ds(r8 + k * HOP, nrows), :],
            wsems[b]).wait()


def _prop_sc(g, e4):
    k = pl.kernel(
        _prop_body,
        out_type=jax.ShapeDtypeStruct((NC, N, D), jnp.float32),
        mesh=_mesh(),
        scratch_types=(
            [pltpu.VMEM((2, C), jnp.int32)] * 6
            + [pltpu.VMEM((C, D), jnp.float32)] * 3
            + [pltpu.VMEM_SHARED((N, D), jnp.float32)]
            + [pltpu.SemaphoreType.DMA] * 12
        ),
    )
    return k(g, e4)


def kernel(x, edge_index, q_emb, t_W, t_b, q_W, q_b):
    row3 = edge_index[0].reshape(NW, CHUNKS_PER_TILE, C)
    col3 = edge_index[1].reshape(NW, CHUNKS_PER_TILE, C)
    e4 = jnp.stack([row3, col3], axis=2)

    degp = _deg_sc(col3).reshape(NC, N).T
    h0, ques, g1 = _encoder_prescale(x, t_W, t_b, q_emb, q_W, q_b, degp)
    a1 = _prop_sc(g1, e4)
    g2 = _blend(degp, a1, g1, h0, final=False)
    a2 = _prop_sc(g2, e4)
    h2 = _blend(degp, a2, g2, h0, final=True)
    return ques, h2
